# initial kernel scaffold (unmeasured)
import jax
import jax.numpy as jnp
from jax import lax
from jax.experimental import pallas as pl
from jax.experimental.pallas import tpu as pltpu

N_DEV = 16
N_EXP = 64
E_PER = N_EXP // N_DEV
N_TOK = 512
D_MODEL = 256
D_OUT = 512
ROWS = N_TOK // N_DEV


def kernel(x, router_W, route_idx, expert_W):
    def body(x_ref, rw_ref, idx_ref, ew_ref, out_ref,
             acc_ref, comm_ref, send_sems, recv_sems):
        q = lax.axis_index("i")
        left = lax.rem(q - 1 + N_DEV, N_DEV)
        right = lax.rem(q + 1, N_DEV)

        xv = x_ref[:, :]
        scores = jnp.dot(xv, rw_ref[:, :], preferred_element_type=jnp.float32)
        s_max = jnp.max(scores, axis=-1, keepdims=True)
        probs = jnp.exp(scores - s_max)
        probs = probs / jnp.sum(probs, axis=-1, keepdims=True)
        e_ids = lax.broadcasted_iota(jnp.int32, (N_TOK, N_EXP), 1)
        idx = idx_ref[:, :]
        mask = (e_ids == idx[:, 0:1]) | (e_ids == idx[:, 1:2])
        w = jnp.where(mask, probs, 0.0)
        w = w / jnp.sum(w, axis=-1, keepdims=True)
        wloc = lax.dynamic_slice(w, (0, q * E_PER), (N_TOK, E_PER))
        acc = jnp.zeros((N_TOK, D_OUT), jnp.float32)
        for j in range(E_PER):
            xw = xv * wloc[:, j][:, None]
            acc = acc + jnp.dot(xw, ew_ref[j], preferred_element_type=jnp.float32)
        acc_ref[:, :] = acc

        barrier_sem = pltpu.get_barrier_semaphore()
        for nbr in (left, right):
            pl.semaphore_signal(barrier_sem, inc=1, device_id=(nbr,),
                                device_id_type=pl.DeviceIdType.MESH)
        pl.semaphore_wait(barrier_sem, 2)

        c0 = lax.rem(q - 1 + N_DEV, N_DEV)
        comm_ref[0, :, :] = acc_ref[pl.ds(c0 * ROWS, ROWS), :]

        for s in range(N_DEV - 1):
            rdma = pltpu.make_async_remote_copy(
                src_ref=comm_ref.at[s],
                dst_ref=comm_ref.at[s + 1],
                send_sem=send_sems.at[s],
                recv_sem=recv_sems.at[s],
                device_id=(right,),
                device_id_type=pl.DeviceIdType.MESH,
            )
            rdma.start()
            rdma.wait()
            c = lax.rem(q - 2 - s + 2 * N_DEV, N_DEV)
            if s < N_DEV - 2:
                comm_ref[s + 1, :, :] = (
                    comm_ref[s + 1, :, :] + acc_ref[pl.ds(c * ROWS, ROWS), :]
                )
            else:
                out_ref[:, :] = (
                    comm_ref[s + 1, :, :] + acc_ref[pl.ds(q * ROWS, ROWS), :]
                )

    return pl.pallas_call(
        body,
        out_shape=jax.ShapeDtypeStruct((ROWS, D_OUT), jnp.float32),
        in_specs=[pl.BlockSpec(memory_space=pltpu.VMEM)] * 4,
        out_specs=pl.BlockSpec(memory_space=pltpu.VMEM),
        scratch_shapes=[
            pltpu.VMEM((N_TOK, D_OUT), jnp.float32),
            pltpu.VMEM((N_DEV, ROWS, D_OUT), jnp.float32),
            pltpu.SemaphoreType.DMA((N_DEV - 1,)),
            pltpu.SemaphoreType.DMA((N_DEV - 1,)),
        ],
        compiler_params=pltpu.CompilerParams(collective_id=0),
    )(x, router_W, route_idx, expert_W)


# baseline (device time: 48590 ns/iter reference)
import jax
import jax.numpy as jnp
from jax import lax
from jax.experimental import pallas as pl
from jax.experimental.pallas import tpu as pltpu

N_DEV = 16
N_EXP = 64
E_PER = N_EXP // N_DEV
N_TOK = 512
D_MODEL = 256
D_OUT = 512
ROWS = N_TOK // N_DEV


def kernel(x, router_W, route_idx, expert_W):
    def body(x_ref, rw_ref, idx_ref, ew_ref, out_ref,
             acc_ref, comm_ref, send_sems, recv_sems):
        q = lax.axis_index("i")
        left = lax.rem(q - 1 + N_DEV, N_DEV)
        right = lax.rem(q + 1, N_DEV)

        xv = x_ref[:, :]
        scores = jnp.dot(xv, rw_ref[:, :], preferred_element_type=jnp.float32)
        s_max = jnp.max(scores, axis=-1, keepdims=True)
        probs = jnp.exp(scores - s_max)
        probs = probs / jnp.sum(probs, axis=-1, keepdims=True)
        e_ids = lax.broadcasted_iota(jnp.int32, (N_TOK, N_EXP), 1)
        idx = idx_ref[:, :]
        mask = (e_ids == idx[:, 0:1]) | (e_ids == idx[:, 1:2])
        w = jnp.where(mask, probs, 0.0)
        w = w / jnp.sum(w, axis=-1, keepdims=True)
        acc = jnp.zeros((N_TOK, D_OUT), jnp.float32)
        for j in range(E_PER):
            e_j = q * E_PER + j
            wcol = jnp.sum(jnp.where(e_ids == e_j, w, 0.0), axis=1)
            xw = xv * wcol[:, None]
            acc = acc + jnp.dot(xw, ew_ref[j], preferred_element_type=jnp.float32)
        acc_ref[:, :] = acc

        barrier_sem = pltpu.get_barrier_semaphore()
        for nbr in (left, right):
            pl.semaphore_signal(barrier_sem, inc=1, device_id=(nbr,),
                                device_id_type=pl.DeviceIdType.MESH)
        pl.semaphore_wait(barrier_sem, 2)

        c0 = lax.rem(q - 1 + N_DEV, N_DEV)
        comm_ref[0, :, :] = acc_ref[pl.ds(c0 * ROWS, ROWS), :]

        for s in range(N_DEV - 1):
            rdma = pltpu.make_async_remote_copy(
                src_ref=comm_ref.at[s],
                dst_ref=comm_ref.at[s + 1],
                send_sem=send_sems.at[s],
                recv_sem=recv_sems.at[s],
                device_id=(right,),
                device_id_type=pl.DeviceIdType.MESH,
            )
            rdma.start()
            rdma.wait()
            c = lax.rem(q - 2 - s + 2 * N_DEV, N_DEV)
            if s < N_DEV - 2:
                comm_ref[s + 1, :, :] = (
                    comm_ref[s + 1, :, :] + acc_ref[pl.ds(c * ROWS, ROWS), :]
                )
            else:
                out_ref[:, :] = (
                    comm_ref[s + 1, :, :] + acc_ref[pl.ds(q * ROWS, ROWS), :]
                )

    return pl.pallas_call(
        body,
        out_shape=jax.ShapeDtypeStruct((ROWS, D_OUT), jnp.float32),
        in_specs=[pl.BlockSpec(memory_space=pltpu.VMEM)] * 4,
        out_specs=pl.BlockSpec(memory_space=pltpu.VMEM),
        scratch_shapes=[
            pltpu.VMEM((N_TOK, D_OUT), jnp.float32),
            pltpu.VMEM((N_DEV, ROWS, D_OUT), jnp.float32),
            pltpu.SemaphoreType.DMA((N_DEV - 1,)),
            pltpu.SemaphoreType.DMA((N_DEV - 1,)),
        ],
        compiler_params=pltpu.CompilerParams(collective_id=0),
    )(x, router_W, route_idx, expert_W)


# device time: 27239 ns/iter; 1.7838x vs baseline; 1.7838x over previous
import jax
import jax.numpy as jnp
from jax import lax
from jax.experimental import pallas as pl
from jax.experimental.pallas import tpu as pltpu

N_DEV = 16
N_EXP = 64
E_PER = N_EXP // N_DEV
N_TOK = 512
D_MODEL = 256
D_OUT = 512
ROWS = N_TOK // N_DEV


def kernel(x, router_W, route_idx, expert_W):
    def body(x_ref, rw_ref, idx_ref, ew_ref, out_ref,
             acc_ref, comm_ref, send_sems, recv_sems):
        q = lax.axis_index("i")

        xv = x_ref[:, :]
        scores = jnp.dot(xv, rw_ref[:, :], preferred_element_type=jnp.float32)
        s_max = jnp.max(scores, axis=-1, keepdims=True)
        probs = jnp.exp(scores - s_max)
        probs = probs / jnp.sum(probs, axis=-1, keepdims=True)
        e_ids = lax.broadcasted_iota(jnp.int32, (N_TOK, N_EXP), 1)
        idx = idx_ref[:, :]
        mask = (e_ids == idx[:, 0:1]) | (e_ids == idx[:, 1:2])
        w = jnp.where(mask, probs, 0.0)
        w = w / jnp.sum(w, axis=-1, keepdims=True)
        acc = jnp.zeros((N_TOK, D_OUT), jnp.float32)
        for j in range(E_PER):
            e_j = q * E_PER + j
            wcol = jnp.sum(jnp.where(e_ids == e_j, w, 0.0), axis=1)
            xw = xv * wcol[:, None]
            acc = acc + jnp.dot(xw, ew_ref[j], preferred_element_type=jnp.float32)
        acc_ref[:, :] = acc

        barrier_sem = pltpu.get_barrier_semaphore()
        for r in range(1, N_DEV):
            p = lax.rem(q + r, N_DEV)
            pl.semaphore_signal(barrier_sem, inc=1, device_id=(p,),
                                device_id_type=pl.DeviceIdType.MESH)
        pl.semaphore_wait(barrier_sem, N_DEV - 1)

        rdmas = []
        for r in range(1, N_DEV):
            p = lax.rem(q - r + N_DEV, N_DEV)
            rdma = pltpu.make_async_remote_copy(
                src_ref=acc_ref.at[pl.ds(p * ROWS, ROWS), :],
                dst_ref=comm_ref.at[r - 1],
                send_sem=send_sems.at[r - 1],
                recv_sem=recv_sems.at[r - 1],
                device_id=(p,),
                device_id_type=pl.DeviceIdType.MESH,
            )
            rdma.start()
            rdmas.append(rdma)
        for rdma in rdmas:
            rdma.wait()

        total = acc_ref[pl.ds(q * ROWS, ROWS), :]
        total = total + jnp.sum(comm_ref[:, :, :], axis=0)
        out_ref[:, :] = total

    return pl.pallas_call(
        body,
        out_shape=jax.ShapeDtypeStruct((ROWS, D_OUT), jnp.float32),
        in_specs=[pl.BlockSpec(memory_space=pltpu.VMEM)] * 4,
        out_specs=pl.BlockSpec(memory_space=pltpu.VMEM),
        scratch_shapes=[
            pltpu.VMEM((N_TOK, D_OUT), jnp.float32),
            pltpu.VMEM((N_DEV - 1, ROWS, D_OUT), jnp.float32),
            pltpu.SemaphoreType.DMA((N_DEV - 1,)),
            pltpu.SemaphoreType.DMA((N_DEV - 1,)),
        ],
        compiler_params=pltpu.CompilerParams(collective_id=0),
    )(x, router_W, route_idx, expert_W)
